# Initial kernel scaffold; baseline (speedup 1.0000x reference)
import jax
import jax.numpy as jnp
from jax import lax
from jax.experimental import pallas as pl
from jax.experimental.pallas import tpu as pltpu
from jax.experimental.pallas import tpu_sc as plsc

N = 10000
E = 320000
D = 128

NPAD = 10240
NW = 32
EB = 128
S = 80
E_PAD = NW * S * EB
R = NPAD // 16
RC = R // 128

_mesh = plsc.VectorSubcoreMesh(core_axis_name="c", subcore_axis_name="s")


def _sc_deg_body(dst_hbm, out_hbm, deg_sh, idx_v, ones_v, buf_v):
    c = lax.axis_index("c")
    s = lax.axis_index("s")
    w = c * 16 + s
    base = s * R

    @pl.loop(0, EB)
    def _init(i):
        ones_v[i] = jnp.ones((16,), jnp.float32)
        buf_v[i] = jnp.zeros((16,), jnp.float32)

    for t in range(RC):
        pltpu.sync_copy(buf_v, deg_sh.at[pl.ds(base + 128 * t, 128)])
    pltpu.sync_copy(dst_hbm.at[w], idx_v)
    plsc.subcore_barrier()

    @pl.loop(0, S)
    def _hist(j):
        pltpu.sync_copy(ones_v, deg_sh.at[idx_v.at[j]], add=True)

    plsc.subcore_barrier()
    for t in range(RC):
        pltpu.sync_copy(deg_sh.at[pl.ds(base + 128 * t, 128)], buf_v)
        pltpu.sync_copy(buf_v, out_hbm.at[c, pl.ds(base + 128 * t, 128)])


_sc_deg = pl.kernel(
    _sc_deg_body,
    out_type=jax.ShapeDtypeStruct((2, NPAD, 16), jnp.float32),
    mesh=_mesh,
    scratch_types=[
        pltpu.VMEM_SHARED((NPAD, 16), jnp.float32),
        pltpu.VMEM((S, EB), jnp.int32),
        pltpu.VMEM((EB, 16), jnp.float32),
        pltpu.VMEM((EB, 16), jnp.float32),
    ],
)


def _sc_agg_body(y_hbm, src_hbm, dst_hbm, out_hbm, acc_sh, isrc_v, idst_v,
                 rows_v, sem):
    c = lax.axis_index("c")
    s = lax.axis_index("s")
    w = c * 16 + s
    base = s * R

    @pl.loop(0, EB)
    def _zero(i):
        for k in range(8):
            rows_v[i, pl.ds(k * 16, 16)] = jnp.zeros((16,), jnp.float32)

    for t in range(RC):
        pltpu.sync_copy(rows_v, acc_sh.at[pl.ds(base + 128 * t, 128)])
    pltpu.sync_copy(src_hbm.at[w], isrc_v)
    pltpu.sync_copy(dst_hbm.at[w], idst_v)
    plsc.subcore_barrier()

    @pl.loop(0, S)
    def _edges(j):
        pltpu.async_copy(y_hbm.at[isrc_v.at[j]], rows_v, sem).wait()
        pltpu.sync_copy(rows_v, acc_sh.at[idst_v.at[j]], add=True)

    plsc.subcore_barrier()
    for t in range(RC):
        pltpu.sync_copy(acc_sh.at[pl.ds(base + 128 * t, 128)], rows_v)
        pltpu.sync_copy(rows_v, out_hbm.at[c, pl.ds(base + 128 * t, 128)])


_sc_agg = pl.kernel(
    _sc_agg_body,
    out_type=jax.ShapeDtypeStruct((2, NPAD, D), jnp.float32),
    mesh=_mesh,
    scratch_types=[
        pltpu.VMEM_SHARED((NPAD, D), jnp.float32),
        pltpu.VMEM((S, EB), jnp.int32),
        pltpu.VMEM((S, EB), jnp.int32),
        pltpu.VMEM((EB, D), jnp.float32),
        pltpu.SemaphoreType.DMA,
    ],
)

BLK = 2560


def _tc_y_body(x_ref, w_ref, degs_ref, y_ref):
    deg = degs_ref[0, :, 0] + degs_ref[1, :, 0] + 1.0
    dis = lax.rsqrt(deg)
    xw = jnp.dot(x_ref[...], w_ref[...], preferred_element_type=jnp.float32)
    y_ref[...] = xw * dis[:, None]


_tc_y = pl.pallas_call(
    _tc_y_body,
    grid=(NPAD // BLK,),
    in_specs=[
        pl.BlockSpec((BLK, D), lambda i: (i, 0)),
        pl.BlockSpec((D, D), lambda i: (0, 0)),
        pl.BlockSpec((2, BLK, 16), lambda i: (0, i, 0)),
    ],
    out_specs=pl.BlockSpec((BLK, D), lambda i: (i, 0)),
    out_shape=jax.ShapeDtypeStruct((NPAD, D), jnp.float32),
)


def _tc_final_body(acc_ref, y_ref, degs_ref, b_ref, o_ref):
    deg = degs_ref[0, :, 0] + degs_ref[1, :, 0] + 1.0
    dis = lax.rsqrt(deg)
    h = acc_ref[0] + acc_ref[1] + y_ref[...]
    h = h * dis[:, None] + b_ref[...]
    nrm = jnp.sqrt(jnp.sum(h * h, axis=1, keepdims=True))
    o_ref[...] = h / jnp.maximum(nrm, 1e-12)


_tc_final = pl.pallas_call(
    _tc_final_body,
    grid=(NPAD // BLK,),
    in_specs=[
        pl.BlockSpec((2, BLK, D), lambda i: (0, i, 0)),
        pl.BlockSpec((BLK, D), lambda i: (i, 0)),
        pl.BlockSpec((2, BLK, 16), lambda i: (0, i, 0)),
        pl.BlockSpec((1, D), lambda i: (0, 0)),
    ],
    out_specs=pl.BlockSpec((BLK, D), lambda i: (i, 0)),
    out_shape=jax.ShapeDtypeStruct((NPAD, D), jnp.float32),
)


@jax.jit
def kernel(x, edge_index, W, b):
    src = edge_index[0]
    dst = edge_index[1]
    pad = N + jnp.arange(E_PAD - E, dtype=jnp.int32) % (NPAD - N)
    src_p = jnp.concatenate([src, pad]).reshape(NW, S, EB)
    dst_p = jnp.concatenate([dst, pad]).reshape(NW, S, EB)
    x_pad = jnp.pad(x, ((0, NPAD - N), (0, 0)))

    degs = _sc_deg(dst_p)
    y = _tc_y(x_pad, W, degs)
    accs = _sc_agg(y, src_p, dst_p)
    out = _tc_final(accs, y, degs, b.reshape(1, D))
    return out[:N]


# speedup vs baseline: 30.2457x; 30.2457x over previous
import functools

import jax
import jax.numpy as jnp
from jax import lax
from jax.experimental import pallas as pl
from jax.experimental.pallas import tpu as pltpu
from jax.experimental.pallas import tpu_sc as plsc

N = 10000
E = 320000
D = 128

NPAD = 10240
NW = 32
EB = 128
S = 80
E_PAD = NW * S * EB
R = NPAD // 16
RC = R // 128

def _sc_deg_body(dst_hbm, out_hbm, deg_sh, idx_v, ones_v, buf_v):
    c = lax.axis_index("c")
    s = lax.axis_index("s")
    w = c * 16 + s
    base = s * R

    @pl.loop(0, EB)
    def _init(i):
        ones_v[i] = jnp.ones((16,), jnp.float32)
        buf_v[i] = jnp.zeros((16,), jnp.float32)

    for t in range(RC):
        pltpu.sync_copy(buf_v, deg_sh.at[pl.ds(base + 128 * t, 128)])
    pltpu.sync_copy(dst_hbm.at[w], idx_v)
    plsc.subcore_barrier()

    @pl.loop(0, S)
    def _hist(j):
        pltpu.sync_copy(ones_v, deg_sh.at[idx_v.at[j]], add=True)

    plsc.subcore_barrier()
    for t in range(RC):
        pltpu.sync_copy(deg_sh.at[pl.ds(base + 128 * t, 128)], buf_v)
        pltpu.sync_copy(buf_v, out_hbm.at[c, pl.ds(base + 128 * t, 128)])


@functools.cache
def _sc_deg():
    mesh = plsc.VectorSubcoreMesh(core_axis_name="c", subcore_axis_name="s")
    return pl.kernel(
        _sc_deg_body,
        out_type=jax.ShapeDtypeStruct((2, NPAD, 16), jnp.float32),
        mesh=mesh,
        scratch_types=[
            pltpu.VMEM_SHARED((NPAD, 16), jnp.float32),
            pltpu.VMEM((S, EB), jnp.int32),
            pltpu.VMEM((EB, 16), jnp.float32),
            pltpu.VMEM((EB, 16), jnp.float32),
        ],
    )


def _sc_agg_body(y_hbm, src_hbm, dst_hbm, out_hbm, acc_sh, isrc_v, idst_v,
                 rows_v, sem):
    c = lax.axis_index("c")
    s = lax.axis_index("s")
    w = c * 16 + s
    base = s * R

    @pl.loop(0, EB)
    def _zero(i):
        for k in range(8):
            rows_v[i, pl.ds(k * 16, 16)] = jnp.zeros((16,), jnp.float32)

    for t in range(RC):
        pltpu.sync_copy(rows_v, acc_sh.at[pl.ds(base + 128 * t, 128)])
    pltpu.sync_copy(src_hbm.at[w], isrc_v)
    pltpu.sync_copy(dst_hbm.at[w], idst_v)
    plsc.subcore_barrier()

    @pl.loop(0, S)
    def _edges(j):
        pltpu.async_copy(y_hbm.at[isrc_v.at[j]], rows_v, sem).wait()
        pltpu.sync_copy(rows_v, acc_sh.at[idst_v.at[j]], add=True)

    plsc.subcore_barrier()
    for t in range(RC):
        pltpu.sync_copy(acc_sh.at[pl.ds(base + 128 * t, 128)], rows_v)
        pltpu.sync_copy(rows_v, out_hbm.at[c, pl.ds(base + 128 * t, 128)])


@functools.cache
def _sc_agg():
    mesh = plsc.VectorSubcoreMesh(core_axis_name="c", subcore_axis_name="s")
    return pl.kernel(
        _sc_agg_body,
        out_type=jax.ShapeDtypeStruct((2, NPAD, D), jnp.float32),
        mesh=mesh,
        scratch_types=[
            pltpu.VMEM_SHARED((NPAD, D), jnp.float32),
            pltpu.VMEM((S, EB), jnp.int32),
            pltpu.VMEM((S, EB), jnp.int32),
            pltpu.VMEM((EB, D), jnp.float32),
            pltpu.SemaphoreType.DMA,
        ],
    )

BLK = 2560


def _tc_y_body(x_ref, w_ref, degs_ref, y_ref):
    deg = degs_ref[0, :, 0] + degs_ref[1, :, 0] + 1.0
    dis = lax.rsqrt(deg)
    xw = jnp.dot(x_ref[...], w_ref[...], preferred_element_type=jnp.float32)
    y_ref[...] = xw * dis[:, None]


_tc_y = pl.pallas_call(
    _tc_y_body,
    grid=(NPAD // BLK,),
    in_specs=[
        pl.BlockSpec((BLK, D), lambda i: (i, 0)),
        pl.BlockSpec((D, D), lambda i: (0, 0)),
        pl.BlockSpec((2, BLK, 16), lambda i: (0, i, 0)),
    ],
    out_specs=pl.BlockSpec((BLK, D), lambda i: (i, 0)),
    out_shape=jax.ShapeDtypeStruct((NPAD, D), jnp.float32),
)


def _tc_final_body(acc_ref, y_ref, degs_ref, b_ref, o_ref):
    deg = degs_ref[0, :, 0] + degs_ref[1, :, 0] + 1.0
    dis = lax.rsqrt(deg)
    h = acc_ref[0] + acc_ref[1] + y_ref[...]
    h = h * dis[:, None] + b_ref[...]
    nrm = jnp.sqrt(jnp.sum(h * h, axis=1, keepdims=True))
    o_ref[...] = h / jnp.maximum(nrm, 1e-12)


_tc_final = pl.pallas_call(
    _tc_final_body,
    grid=(NPAD // BLK,),
    in_specs=[
        pl.BlockSpec((2, BLK, D), lambda i: (0, i, 0)),
        pl.BlockSpec((BLK, D), lambda i: (i, 0)),
        pl.BlockSpec((2, BLK, 16), lambda i: (0, i, 0)),
        pl.BlockSpec((1, D), lambda i: (0, 0)),
    ],
    out_specs=pl.BlockSpec((BLK, D), lambda i: (i, 0)),
    out_shape=jax.ShapeDtypeStruct((NPAD, D), jnp.float32),
)


@jax.jit
def kernel(x, edge_index, W, b):
    src = edge_index[0]
    dst = edge_index[1]
    pad = N + jnp.arange(E_PAD - E, dtype=jnp.int32) % (NPAD - N)
    src_p = jnp.concatenate([src, pad]).reshape(NW, S, EB)
    dst_p = jnp.concatenate([dst, pad]).reshape(NW, S, EB)
    x_pad = jnp.pad(x, ((0, NPAD - N), (0, 0)))

    degs = _sc_deg()(dst_p)
    y = _tc_y(x_pad, W, degs)
    accs = _sc_agg()(y, src_p, dst_p)
    out = _tc_final(accs, y, degs, b.reshape(1, D))
    return out[:N]


# speedup vs baseline: 38.6221x; 1.2769x over previous
import functools

import jax
import jax.numpy as jnp
from jax import lax
from jax.experimental import pallas as pl
from jax.experimental.pallas import tpu as pltpu
from jax.experimental.pallas import tpu_sc as plsc

N = 10000
E = 320000
D = 128

NPAD = 10240
NW = 32
EB = 128
S = 80
E_PAD = NW * S * EB
R = NPAD // 16
CH = 128
NCH = R // CH
Q = 16
NQ = S // Q

def _sc_deg_body(dst_hbm, out_hbm, deg_sh, idx_v, ones_v, buf_v):
    c = lax.axis_index("c")
    s = lax.axis_index("s")
    w = c * 16 + s
    base = s * R

    @pl.loop(0, EB)
    def _init(i):
        ones_v[i] = jnp.ones((16,), jnp.float32)
        buf_v[i] = jnp.zeros((16,), jnp.float32)

    for t in range(NCH):
        pltpu.sync_copy(buf_v.at[pl.ds(0, CH)], deg_sh.at[pl.ds(base + CH * t, CH)])
    pltpu.sync_copy(dst_hbm.at[w], idx_v)
    plsc.subcore_barrier()

    @pl.loop(0, S)
    def _hist(j):
        pltpu.sync_copy(ones_v, deg_sh.at[idx_v.at[j]], add=True)

    plsc.subcore_barrier()
    for t in range(NCH):
        pltpu.sync_copy(deg_sh.at[pl.ds(base + CH * t, CH)], buf_v.at[pl.ds(0, CH)])
        pltpu.sync_copy(buf_v.at[pl.ds(0, CH)], out_hbm.at[c, pl.ds(base + CH * t, CH)])


@functools.cache
def _sc_deg():
    mesh = plsc.VectorSubcoreMesh(core_axis_name="c", subcore_axis_name="s")
    return pl.kernel(
        _sc_deg_body,
        out_type=jax.ShapeDtypeStruct((2, NPAD, 16), jnp.float32),
        mesh=mesh,
        scratch_types=[
            pltpu.VMEM_SHARED((NPAD, 16), jnp.float32),
            pltpu.VMEM((S, EB), jnp.int32),
            pltpu.VMEM((EB, 16), jnp.float32),
            pltpu.VMEM((EB, 16), jnp.float32),
        ],
    )


def _sc_agg_body(y_hbm, src_hbm, dst_hbm, out_hbm, acc_sh, isrc_v, idst_v,
                 rows0_v, rows1_v, sem0, sem1):
    c = lax.axis_index("c")
    s = lax.axis_index("s")
    w = c * 16 + s
    base = s * R

    @pl.loop(0, EB)
    def _zero(i):
        for k in range(8):
            rows0_v[i, pl.ds(k * 16, 16)] = jnp.zeros((16,), jnp.float32)

    for t in range(NCH):
        pltpu.sync_copy(rows0_v.at[pl.ds(0, CH)], acc_sh.at[pl.ds(base + CH * t, CH)])
    plsc.subcore_barrier()

    for q in range(NQ):
        pltpu.sync_copy(src_hbm.at[w, pl.ds(q * Q, Q)], isrc_v)
        pltpu.sync_copy(dst_hbm.at[w, pl.ds(q * Q, Q)], idst_v)
        pltpu.async_copy(y_hbm.at[isrc_v.at[0]], rows0_v, sem0)

        @pl.loop(0, Q, step=2)
        def _edges(j):
            pltpu.async_copy(y_hbm.at[isrc_v.at[j + 1]], rows1_v, sem1)
            pltpu.make_async_copy(y_hbm.at[isrc_v.at[j]], rows0_v, sem0).wait()
            pltpu.sync_copy(rows0_v, acc_sh.at[idst_v.at[j]], add=True)
            pltpu.async_copy(y_hbm.at[isrc_v.at[lax.rem(j + 2, Q)]], rows0_v, sem0)
            pltpu.make_async_copy(y_hbm.at[isrc_v.at[j + 1]], rows1_v, sem1).wait()
            pltpu.sync_copy(rows1_v, acc_sh.at[idst_v.at[j + 1]], add=True)

        pltpu.make_async_copy(y_hbm.at[isrc_v.at[0]], rows0_v, sem0).wait()

    plsc.subcore_barrier()
    for t in range(NCH):
        pltpu.sync_copy(acc_sh.at[pl.ds(base + CH * t, CH)], rows0_v.at[pl.ds(0, CH)])
        pltpu.sync_copy(rows0_v.at[pl.ds(0, CH)], out_hbm.at[c, pl.ds(base + CH * t, CH)])


@functools.cache
def _sc_agg():
    mesh = plsc.VectorSubcoreMesh(core_axis_name="c", subcore_axis_name="s")
    return pl.kernel(
        _sc_agg_body,
        out_type=jax.ShapeDtypeStruct((2, NPAD, D), jnp.float32),
        mesh=mesh,
        scratch_types=[
            pltpu.VMEM_SHARED((NPAD, D), jnp.float32),
            pltpu.VMEM((Q, EB), jnp.int32),
            pltpu.VMEM((Q, EB), jnp.int32),
            pltpu.VMEM((EB, D), jnp.float32),
            pltpu.VMEM((EB, D), jnp.float32),
            pltpu.SemaphoreType.DMA,
            pltpu.SemaphoreType.DMA,
        ],
    )

BLK = 2560


def _tc_y_body(x_ref, w_ref, degs_ref, y_ref):
    deg = degs_ref[0, :, 0] + degs_ref[1, :, 0] + 1.0
    dis = lax.rsqrt(deg)
    xw = jnp.dot(x_ref[...], w_ref[...], preferred_element_type=jnp.float32)
    y_ref[...] = xw * dis[:, None]


_tc_y = pl.pallas_call(
    _tc_y_body,
    grid=(NPAD // BLK,),
    in_specs=[
        pl.BlockSpec((BLK, D), lambda i: (i, 0)),
        pl.BlockSpec((D, D), lambda i: (0, 0)),
        pl.BlockSpec((2, BLK, 16), lambda i: (0, i, 0)),
    ],
    out_specs=pl.BlockSpec((BLK, D), lambda i: (i, 0)),
    out_shape=jax.ShapeDtypeStruct((NPAD, D), jnp.float32),
)


def _tc_final_body(acc_ref, y_ref, degs_ref, b_ref, o_ref):
    deg = degs_ref[0, :, 0] + degs_ref[1, :, 0] + 1.0
    dis = lax.rsqrt(deg)
    h = acc_ref[0] + acc_ref[1] + y_ref[...]
    h = h * dis[:, None] + b_ref[...]
    nrm = jnp.sqrt(jnp.sum(h * h, axis=1, keepdims=True))
    o_ref[...] = h / jnp.maximum(nrm, 1e-12)


_tc_final = pl.pallas_call(
    _tc_final_body,
    grid=(NPAD // BLK,),
    in_specs=[
        pl.BlockSpec((2, BLK, D), lambda i: (0, i, 0)),
        pl.BlockSpec((BLK, D), lambda i: (i, 0)),
        pl.BlockSpec((2, BLK, 16), lambda i: (0, i, 0)),
        pl.BlockSpec((1, D), lambda i: (0, 0)),
    ],
    out_specs=pl.BlockSpec((BLK, D), lambda i: (i, 0)),
    out_shape=jax.ShapeDtypeStruct((NPAD, D), jnp.float32),
)


@jax.jit
def kernel(x, edge_index, W, b):
    src = edge_index[0]
    dst = edge_index[1]
    pad = N + jnp.arange(E_PAD - E, dtype=jnp.int32) % (NPAD - N)
    src_p = jnp.concatenate([src, pad]).reshape(NW, S, EB)
    dst_p = jnp.concatenate([dst, pad]).reshape(NW, S, EB)
    x_pad = jnp.pad(x, ((0, NPAD - N), (0, 0)))

    degs = _sc_deg()(dst_p)
    y = _tc_y(x_pad, W, degs)
    accs = _sc_agg()(y, src_p, dst_p)
    out = _tc_final(accs, y, degs, b.reshape(1, D))
    return out[:N]
